# baseline (speedup 1.0000x reference)
import functools

import jax
import jax.numpy as jnp
from jax import lax
from jax.experimental import pallas as pl
from jax.experimental.pallas import tpu as pltpu
from jax.experimental.pallas import tpu_sc as plsc

_N = 10000
_NP = 10240
_D = 128
_E = 320000
_Z = 64

_NC = 2
_NS = 16
_NW = _NC * _NS
_EPW = _E // _NW
_CH = 64
_EPWP = 10240
_NCHUNK = _EPWP // _CH
_RPT = _NP // _NS

_mesh = plsc.VectorSubcoreMesh(core_axis_name="c", subcore_axis_name="s")


@functools.partial(
    pl.kernel,
    mesh=_mesh,
    out_type=jax.ShapeDtypeStruct((_NC, _NP, _D), jnp.float32),
    scratch_types=[
        pltpu.VMEM((_NCHUNK, _CH), jnp.int32),
        pltpu.VMEM((_CH, _D), jnp.float32),
        pltpu.VMEM_SHARED((_NP, _D), jnp.float32),
    ],
)
def _deg_kernel(dst_hbm, ones_hbm, zeros_hbm, out_hbm, dst_v, ones_v, deg_sh):
    c = lax.axis_index("c")
    s = lax.axis_index("s")
    wid = s * _NC + c
    row0 = s * _RPT
    pltpu.sync_copy(dst_hbm.at[wid], dst_v)
    pltpu.sync_copy(ones_hbm, ones_v)
    pltpu.sync_copy(zeros_hbm.at[pl.ds(row0, _RPT)], deg_sh.at[pl.ds(row0, _RPT)])
    plsc.subcore_barrier()

    def body(j, carry):
        pltpu.sync_copy(ones_v, deg_sh.at[dst_v.at[j]], add=True)
        return carry

    lax.fori_loop(0, _NCHUNK, body, 0)
    plsc.subcore_barrier()
    pltpu.sync_copy(deg_sh.at[pl.ds(row0, _RPT)], out_hbm.at[c, pl.ds(row0, _RPT)])


@functools.partial(
    pl.kernel,
    mesh=_mesh,
    out_type=jax.ShapeDtypeStruct((_NC, _NP, _D), jnp.float32),
    scratch_types=[
        pltpu.VMEM((_EPWP,), jnp.int32),
        pltpu.VMEM((_NCHUNK, _CH), jnp.int32),
        pltpu.VMEM((_CH, _D), jnp.float32),
        pltpu.VMEM((_CH, _D), jnp.float32),
        pltpu.VMEM_SHARED((_NP, _D), jnp.float32),
        pltpu.SemaphoreType.DMA,
        pltpu.SemaphoreType.DMA,
    ],
)
def _prop_kernel(feats_hbm, src_hbm, dst_hbm, zeros_hbm, out_hbm,
                 src_v, dst_v, rows_a, rows_b, agg_sh, sem_a, sem_b):
    c = lax.axis_index("c")
    s = lax.axis_index("s")
    wid = s * _NC + c
    row0 = s * _RPT
    pltpu.sync_copy(src_hbm.at[wid], src_v)
    pltpu.sync_copy(dst_hbm.at[wid], dst_v)
    pltpu.sync_copy(zeros_hbm.at[pl.ds(row0, _RPT)], agg_sh.at[pl.ds(row0, _RPT)])
    plsc.subcore_barrier()

    def _sidx(j):
        return src_v.at[pl.ds(j * _CH, _CH)]

    pltpu.async_copy(feats_hbm.at[_sidx(0)], rows_a, sem_a)

    def body(i, carry):
        j0 = i * 2
        pltpu.async_copy(feats_hbm.at[_sidx(j0 + 1)], rows_b, sem_b)
        pltpu.make_async_copy(feats_hbm.at[_sidx(j0)], rows_a, sem_a).wait()
        pltpu.sync_copy(rows_a, agg_sh.at[dst_v.at[j0]], add=True)

        @pl.when(j0 + 2 < _NCHUNK)
        def _():
            pltpu.async_copy(feats_hbm.at[_sidx(j0 + 2)], rows_a, sem_a)

        pltpu.make_async_copy(feats_hbm.at[_sidx(j0 + 1)], rows_b, sem_b).wait()
        pltpu.sync_copy(rows_b, agg_sh.at[dst_v.at[j0 + 1]], add=True)
        return carry

    lax.fori_loop(0, _NCHUNK // 2, body, 0)
    plsc.subcore_barrier()
    pltpu.sync_copy(agg_sh.at[pl.ds(row0, _RPT)], out_hbm.at[c, pl.ds(row0, _RPT)])


_BT = 1024


def _scale_body(x_ref, degp_ref, xs_ref, invs_ref, invd_ref):
    deg = degp_ref[0, :, 0:1] + degp_ref[1, :, 0:1] + 1.0
    invs = lax.rsqrt(deg)
    invd = 1.0 / deg
    xs_ref[...] = x_ref[...] * invs
    invs_ref[...] = jnp.broadcast_to(invs, (_BT, 8))
    invd_ref[...] = jnp.broadcast_to(invd, (_BT, 8))


def _hidden_body(x_ref, agg_ref, invs_ref, invd_ref, w_ref, b_ref,
                 hid_ref, hs_ref):
    invs = invs_ref[:, 0:1]
    invd = invd_ref[:, 0:1]
    prop = (agg_ref[0] + agg_ref[1]) * invs + x_ref[...] * invd
    h = jnp.dot(prop, w_ref[...], preferred_element_type=jnp.float32,
                precision=lax.Precision.HIGHEST)
    h = jnp.maximum(h + b_ref[...], 0.0)
    hid_ref[...] = h
    hs_ref[...] = h * invs


def _head_body(hid_ref, agg_ref, invs_ref, invd_ref, w_ref, b_ref,
               mu_ref, lv_ref):
    invs = invs_ref[:, 0:1]
    invd = invd_ref[:, 0:1]
    prop = (agg_ref[0] + agg_ref[1]) * invs + hid_ref[...] * invd
    o = jnp.dot(prop, w_ref[...], preferred_element_type=jnp.float32,
                precision=lax.Precision.HIGHEST) + b_ref[...]
    mu_ref[...] = o[:, :_Z]
    lv_ref[...] = o[:, _Z:]


def _scale_call(x, degp):
    return pl.pallas_call(
        _scale_body,
        grid=(_NP // _BT,),
        in_specs=[
            pl.BlockSpec((_BT, _D), lambda i: (i, 0)),
            pl.BlockSpec((_NC, _BT, _D), lambda i: (0, i, 0)),
        ],
        out_specs=[
            pl.BlockSpec((_BT, _D), lambda i: (i, 0)),
            pl.BlockSpec((_BT, 8), lambda i: (i, 0)),
            pl.BlockSpec((_BT, 8), lambda i: (i, 0)),
        ],
        out_shape=[
            jax.ShapeDtypeStruct((_NP, _D), jnp.float32),
            jax.ShapeDtypeStruct((_NP, 8), jnp.float32),
            jax.ShapeDtypeStruct((_NP, 8), jnp.float32),
        ],
    )(x, degp)


def _hidden_call(x, agg, invs8, invd8, w, b):
    return pl.pallas_call(
        _hidden_body,
        grid=(_NP // _BT,),
        in_specs=[
            pl.BlockSpec((_BT, _D), lambda i: (i, 0)),
            pl.BlockSpec((_NC, _BT, _D), lambda i: (0, i, 0)),
            pl.BlockSpec((_BT, 8), lambda i: (i, 0)),
            pl.BlockSpec((_BT, 8), lambda i: (i, 0)),
            pl.BlockSpec((_D, _D), lambda i: (0, 0)),
            pl.BlockSpec((1, _D), lambda i: (0, 0)),
        ],
        out_specs=[
            pl.BlockSpec((_BT, _D), lambda i: (i, 0)),
            pl.BlockSpec((_BT, _D), lambda i: (i, 0)),
        ],
        out_shape=[
            jax.ShapeDtypeStruct((_NP, _D), jnp.float32),
            jax.ShapeDtypeStruct((_NP, _D), jnp.float32),
        ],
    )(x, agg, invs8, invd8, w, b)


def _head_call(hid, agg, invs8, invd8, w, b):
    return pl.pallas_call(
        _head_body,
        grid=(_NP // _BT,),
        in_specs=[
            pl.BlockSpec((_BT, _D), lambda i: (i, 0)),
            pl.BlockSpec((_NC, _BT, _D), lambda i: (0, i, 0)),
            pl.BlockSpec((_BT, 8), lambda i: (i, 0)),
            pl.BlockSpec((_BT, 8), lambda i: (i, 0)),
            pl.BlockSpec((_D, _D), lambda i: (0, 0)),
            pl.BlockSpec((1, _D), lambda i: (0, 0)),
        ],
        out_specs=[
            pl.BlockSpec((_BT, _Z), lambda i: (i, 0)),
            pl.BlockSpec((_BT, _Z), lambda i: (i, 0)),
        ],
        out_shape=[
            jax.ShapeDtypeStruct((_NP, _Z), jnp.float32),
            jax.ShapeDtypeStruct((_NP, _Z), jnp.float32),
        ],
    )(hid, agg, invs8, invd8, w, b)


def kernel(x, edge_index, W_h, b_h, W_mu, b_mu, W_std, b_std):
    src2 = jnp.pad(edge_index[0].reshape(_NW, _EPW), ((0, 0), (0, _EPWP - _EPW)),
                   constant_values=_N)
    dst2 = jnp.pad(edge_index[1].reshape(_NW, _EPW), ((0, 0), (0, _EPWP - _EPW)),
                   constant_values=_N)
    dst3 = dst2.reshape(_NW, _NCHUNK, _CH)
    xp = jnp.zeros((_NP, _D), jnp.float32).at[:_N].set(x)
    zeros_nd = jnp.zeros((_NP, _D), jnp.float32)
    ones_chd = jnp.ones((_CH, _D), jnp.float32)

    degp = _deg_kernel(dst3, ones_chd, zeros_nd)
    xs, invs8, invd8 = _scale_call(xp, degp)
    agg1 = _prop_kernel(xs, src2, dst3, zeros_nd)
    hidden, hs = _hidden_call(xp, agg1, invs8, invd8, W_h, b_h.reshape(1, _D))
    agg2 = _prop_kernel(hs, src2, dst3, zeros_nd)
    w_cat = jnp.concatenate([W_mu, W_std], axis=1)
    b_cat = jnp.concatenate([b_mu, b_std]).reshape(1, _D)
    mu, logvar = _head_call(hidden, agg2, invs8, invd8, w_cat, b_cat)
    return (mu[:_N], logvar[:_N])


# speedup vs baseline: 1.0770x; 1.0770x over previous
import functools

import jax
import jax.numpy as jnp
from jax import lax
from jax.experimental import pallas as pl
from jax.experimental.pallas import tpu as pltpu
from jax.experimental.pallas import tpu_sc as plsc

_N = 10000
_NP = 10240
_D = 128
_E = 320000
_Z = 64

_NC = 2
_NS = 16
_NW = _NC * _NS
_EPW = _E // _NW
_CH = 64
_EPWP = 10176
_NCHUNK = _EPWP // _CH
_RPT = _NP // _NS

_mesh = plsc.VectorSubcoreMesh(core_axis_name="c", subcore_axis_name="s")


@functools.partial(
    pl.kernel,
    mesh=_mesh,
    out_type=jax.ShapeDtypeStruct((_NC, _NP, _D), jnp.float32),
    scratch_types=[
        pltpu.VMEM((_NCHUNK, _CH), jnp.int32),
        pltpu.VMEM((_CH, _D), jnp.float32),
        pltpu.VMEM_SHARED((_NP, _D), jnp.float32),
    ],
)
def _deg_kernel(dst_hbm, ones_hbm, zeros_hbm, out_hbm, dst_v, ones_v, deg_sh):
    c = lax.axis_index("c")
    s = lax.axis_index("s")
    wid = s * _NC + c
    row0 = s * _RPT
    pltpu.sync_copy(dst_hbm.at[wid], dst_v)
    pltpu.sync_copy(ones_hbm, ones_v)
    pltpu.sync_copy(zeros_hbm.at[pl.ds(row0, _RPT)], deg_sh.at[pl.ds(row0, _RPT)])
    plsc.subcore_barrier()

    def body(j, carry):
        pltpu.sync_copy(ones_v, deg_sh.at[dst_v.at[j]], add=True)
        return carry

    lax.fori_loop(0, _NCHUNK, body, 0)
    plsc.subcore_barrier()
    pltpu.sync_copy(deg_sh.at[pl.ds(row0, _RPT)], out_hbm.at[c, pl.ds(row0, _RPT)])


@functools.partial(
    pl.kernel,
    mesh=_mesh,
    out_type=jax.ShapeDtypeStruct((_NC, _NP, _D), jnp.float32),
    scratch_types=[
        [pltpu.VMEM((1, _CH), jnp.int32) for _ in range(3)],
        pltpu.VMEM((_NCHUNK, _CH), jnp.int32),
        [pltpu.VMEM((_CH, _D), jnp.float32) for _ in range(3)],
        pltpu.VMEM_SHARED((_NP, _D), jnp.float32),
        [pltpu.SemaphoreType.DMA for _ in range(3)],
        [pltpu.SemaphoreType.DMA for _ in range(3)],
        [pltpu.SemaphoreType.DMA for _ in range(3)],
    ],
)
def _prop_kernel(feats_hbm, src_hbm, dst_hbm, zeros_hbm, out_hbm,
                 si, dst_v, rows, agg_sh, sem_i, sem_g, sem_s):
    c = lax.axis_index("c")
    s = lax.axis_index("s")
    wid = s * _NC + c
    row0 = s * _RPT
    pltpu.sync_copy(dst_hbm.at[wid], dst_v)
    pltpu.sync_copy(zeros_hbm.at[pl.ds(row0, _RPT)], agg_sh.at[pl.ds(row0, _RPT)])
    plsc.subcore_barrier()

    def _fetch_idx(j, b):
        pltpu.async_copy(src_hbm.at[wid, pl.ds(j, 1)], si[b], sem_i[b])

    def _wait_idx(j, b):
        pltpu.make_async_copy(src_hbm.at[wid, pl.ds(j, 1)], si[b],
                              sem_i[b]).wait()

    def _gather(b):
        pltpu.async_copy(feats_hbm.at[si[b].at[0]], rows[b], sem_g[b])

    def _wait_gather(b):
        pltpu.make_async_copy(feats_hbm.at[si[b].at[0]], rows[b],
                              sem_g[b]).wait()

    def _scat(j, b):
        pltpu.async_copy(rows[b], agg_sh.at[dst_v.at[j]], sem_s[b],
                         add=True)

    def _wait_scat(j, b):
        pltpu.make_async_copy(rows[b], agg_sh.at[dst_v.at[j]],
                              sem_s[b]).wait()

    for b in range(3):
        _fetch_idx(b, b)
    _wait_idx(0, 0)
    _gather(0)

    def body(i, carry):
        for b in range(3):
            j = i * 3 + b
            bn = (b + 1) % 3
            jn = j + 1

            _wait_gather(b)

            @pl.when(j + 3 < _NCHUNK)
            def _():
                _fetch_idx(j + 3, b)

            _scat(j, b)

            @pl.when(jn < _NCHUNK)
            def _():
                @pl.when(jn >= 3)
                def _():
                    _wait_scat(jn - 3, bn)
                _wait_idx(jn, bn)
                _gather(bn)
        return carry

    lax.fori_loop(0, _NCHUNK // 3, body, 0)
    for b in range(3):
        _wait_scat(_NCHUNK - 3 + b, b)
    plsc.subcore_barrier()
    pltpu.sync_copy(agg_sh.at[pl.ds(row0, _RPT)], out_hbm.at[c, pl.ds(row0, _RPT)])


_BT = 1024


def _scale_body(x_ref, degp_ref, xs_ref, invs_ref, invd_ref):
    deg = degp_ref[0, :, 0:1] + degp_ref[1, :, 0:1] + 1.0
    invs = lax.rsqrt(deg)
    invd = 1.0 / deg
    xs_ref[...] = x_ref[...] * invs
    invs_ref[...] = jnp.broadcast_to(invs, (_BT, 8))
    invd_ref[...] = jnp.broadcast_to(invd, (_BT, 8))


def _hidden_body(x_ref, agg_ref, invs_ref, invd_ref, w_ref, b_ref,
                 hid_ref, hs_ref):
    invs = invs_ref[:, 0:1]
    invd = invd_ref[:, 0:1]
    prop = (agg_ref[0] + agg_ref[1]) * invs + x_ref[...] * invd
    h = jnp.dot(prop, w_ref[...], preferred_element_type=jnp.float32,
                precision=lax.Precision.HIGHEST)
    h = jnp.maximum(h + b_ref[...], 0.0)
    hid_ref[...] = h
    hs_ref[...] = h * invs


def _head_body(hid_ref, agg_ref, invs_ref, invd_ref, w_ref, b_ref,
               mu_ref, lv_ref):
    invs = invs_ref[:, 0:1]
    invd = invd_ref[:, 0:1]
    prop = (agg_ref[0] + agg_ref[1]) * invs + hid_ref[...] * invd
    o = jnp.dot(prop, w_ref[...], preferred_element_type=jnp.float32,
                precision=lax.Precision.HIGHEST) + b_ref[...]
    mu_ref[...] = o[:, :_Z]
    lv_ref[...] = o[:, _Z:]


def _scale_call(x, degp):
    return pl.pallas_call(
        _scale_body,
        grid=(_NP // _BT,),
        in_specs=[
            pl.BlockSpec((_BT, _D), lambda i: (i, 0)),
            pl.BlockSpec((_NC, _BT, _D), lambda i: (0, i, 0)),
        ],
        out_specs=[
            pl.BlockSpec((_BT, _D), lambda i: (i, 0)),
            pl.BlockSpec((_BT, 8), lambda i: (i, 0)),
            pl.BlockSpec((_BT, 8), lambda i: (i, 0)),
        ],
        out_shape=[
            jax.ShapeDtypeStruct((_NP, _D), jnp.float32),
            jax.ShapeDtypeStruct((_NP, 8), jnp.float32),
            jax.ShapeDtypeStruct((_NP, 8), jnp.float32),
        ],
    )(x, degp)


def _hidden_call(x, agg, invs8, invd8, w, b):
    return pl.pallas_call(
        _hidden_body,
        grid=(_NP // _BT,),
        in_specs=[
            pl.BlockSpec((_BT, _D), lambda i: (i, 0)),
            pl.BlockSpec((_NC, _BT, _D), lambda i: (0, i, 0)),
            pl.BlockSpec((_BT, 8), lambda i: (i, 0)),
            pl.BlockSpec((_BT, 8), lambda i: (i, 0)),
            pl.BlockSpec((_D, _D), lambda i: (0, 0)),
            pl.BlockSpec((1, _D), lambda i: (0, 0)),
        ],
        out_specs=[
            pl.BlockSpec((_BT, _D), lambda i: (i, 0)),
            pl.BlockSpec((_BT, _D), lambda i: (i, 0)),
        ],
        out_shape=[
            jax.ShapeDtypeStruct((_NP, _D), jnp.float32),
            jax.ShapeDtypeStruct((_NP, _D), jnp.float32),
        ],
    )(x, agg, invs8, invd8, w, b)


def _head_call(hid, agg, invs8, invd8, w, b):
    return pl.pallas_call(
        _head_body,
        grid=(_NP // _BT,),
        in_specs=[
            pl.BlockSpec((_BT, _D), lambda i: (i, 0)),
            pl.BlockSpec((_NC, _BT, _D), lambda i: (0, i, 0)),
            pl.BlockSpec((_BT, 8), lambda i: (i, 0)),
            pl.BlockSpec((_BT, 8), lambda i: (i, 0)),
            pl.BlockSpec((_D, _D), lambda i: (0, 0)),
            pl.BlockSpec((1, _D), lambda i: (0, 0)),
        ],
        out_specs=[
            pl.BlockSpec((_BT, _Z), lambda i: (i, 0)),
            pl.BlockSpec((_BT, _Z), lambda i: (i, 0)),
        ],
        out_shape=[
            jax.ShapeDtypeStruct((_NP, _Z), jnp.float32),
            jax.ShapeDtypeStruct((_NP, _Z), jnp.float32),
        ],
    )(hid, agg, invs8, invd8, w, b)


def kernel(x, edge_index, W_h, b_h, W_mu, b_mu, W_std, b_std):
    src2 = jnp.pad(edge_index[0].reshape(_NW, _EPW), ((0, 0), (0, _EPWP - _EPW)),
                   constant_values=_N)
    dst2 = jnp.pad(edge_index[1].reshape(_NW, _EPW), ((0, 0), (0, _EPWP - _EPW)),
                   constant_values=_N)
    src3 = src2.reshape(_NW, _NCHUNK, _CH)
    dst3 = dst2.reshape(_NW, _NCHUNK, _CH)
    xp = jnp.zeros((_NP, _D), jnp.float32).at[:_N].set(x)
    zeros_nd = jnp.zeros((_NP, _D), jnp.float32)
    ones_chd = jnp.ones((_CH, _D), jnp.float32)

    degp = _deg_kernel(dst3, ones_chd, zeros_nd)
    xs, invs8, invd8 = _scale_call(xp, degp)
    agg1 = _prop_kernel(xs, src3, dst3, zeros_nd)
    hidden, hs = _hidden_call(xp, agg1, invs8, invd8, W_h, b_h.reshape(1, _D))
    agg2 = _prop_kernel(hs, src3, dst3, zeros_nd)
    w_cat = jnp.concatenate([W_mu, W_std], axis=1)
    b_cat = jnp.concatenate([b_mu, b_std]).reshape(1, _D)
    mu, logvar = _head_call(hidden, agg2, invs8, invd8, w_cat, b_cat)
    return (mu[:_N], logvar[:_N])
